# baseline (speedup 1.0000x reference)
import functools

import jax
import jax.numpy as jnp
from jax import lax
from jax.experimental import pallas as pl
from jax.experimental.pallas import tpu as pltpu
from jax.experimental.pallas import tpu_sc as plsc

B = 4096
L = 200
I = 128
X1 = 57

NC = 2
NS = 16
NW = NC * NS
BPW = B // NW
NCH = I // 16
NG = BPW // 16
HALF = NG // 2


def _tree_sum(vals):
    vals = list(vals)
    while len(vals) > 1:
        nxt = [vals[i] + vals[i + 1] for i in range(0, len(vals) - 1, 2)]
        if len(vals) % 2:
            nxt.append(vals[-1])
        vals = nxt
    return vals[0]


@functools.partial(
    pl.kernel,
    mesh=plsc.VectorSubcoreMesh(core_axis_name="c", subcore_axis_name="s"),
    out_type=jax.ShapeDtypeStruct((B,), jnp.float32),
    scratch_types=[
        pltpu.VMEM((BPW, I), jnp.float32),
        pltpu.VMEM((I,), jnp.float32),
        pltpu.VMEM((16,), jnp.float32),
        pltpu.VMEM((16, 17), jnp.float32),
        pltpu.VMEM((BPW,), jnp.float32),
        pltpu.SemaphoreType.DMA,
        pltpu.SemaphoreType.DMA,
    ],
    compiler_params=pltpu.CompilerParams(
        needs_layout_passes=False,
        disable_bounds_checks=True,
        disable_semaphore_checks=True,
    ),
)
def _sc_head(x_hbm, w_hbm, b_hbm, out_hbm,
             rows_v, w_v, b_v, pt_v, out_v, sem0, sem1):
    wid = lax.axis_index("s") * NC + lax.axis_index("c")
    base = wid * BPW
    half_rows = BPW // 2

    cp0 = pltpu.async_copy(
        x_hbm.at[pl.ds(base, half_rows), X1],
        rows_v.at[pl.ds(0, half_rows)], sem0)
    cp1 = pltpu.async_copy(
        x_hbm.at[pl.ds(base + half_rows, half_rows), X1],
        rows_v.at[pl.ds(half_rows, half_rows)], sem1)
    pltpu.sync_copy(w_hbm, w_v)
    pltpu.sync_copy(b_hbm, b_v)

    wc = [w_v[pl.ds(c * 16, 16)] for c in range(NCH)]
    bias_vec = b_v[...]
    iota = lax.broadcasted_iota(jnp.int32, (16,), 0)

    def group_body(g, _):
        def row_body(r, _):
            row = g * 16 + r
            acc = _tree_sum(
                rows_v[row, pl.ds(c * 16, 16)] * wc[c] for c in range(NCH))
            pt_v[r, pl.ds(0, 16)] = acc
            return 0

        lax.fori_loop(0, 16, row_body, 0, unroll=4)

        cols = [
            plsc.load_gather(pt_v, [iota, jnp.full((16,), j, jnp.int32)])
            for j in range(16)
        ]
        res = bias_vec + _tree_sum(cols)
        out_v[pl.ds(g * 16, 16)] = 1.0 / (1.0 + jnp.exp(-res))
        return 0

    cp0.wait()
    lax.fori_loop(0, HALF, group_body, 0)
    cp1.wait()
    lax.fori_loop(HALF, NG, group_body, 0)

    pltpu.sync_copy(out_v, out_hbm.at[pl.ds(base, BPW)])


def kernel(x, W, bias):
    w = W.reshape(I)
    b16 = jnp.broadcast_to(bias, (16,)).astype(jnp.float32)
    probs = _sc_head(x, w, b16)
    return probs.reshape(B, 1, 1)
